# baseline (speedup 1.0000x reference)
import functools
import math

import jax
import jax.numpy as jnp
from jax import lax
from jax.experimental import pallas as pl
from jax.experimental.pallas import tpu as pltpu
from jax.experimental.pallas import tpu_sc as plsc

D = 64
HW = 32
CH = 128
NW = 32
NB = 4
_SCALE = math.sqrt(D)


@functools.lru_cache(maxsize=None)
def _make_kernel(B):
    NCH = 2 * B // (NW * CH)
    R = NCH // NB
    assert R * NB == NCH and R >= 2
    mesh = plsc.VectorSubcoreMesh(core_axis_name="c", subcore_axis_name="s")

    @functools.partial(
        pl.kernel,
        mesh=mesh,
        out_type=jax.ShapeDtypeStruct((2 * B, HW), jnp.float32),
        compiler_params=pltpu.CompilerParams(use_tc_tiling_on_sc=False),
        scratch_types=[
            pltpu.VMEM((NCH, CH), jnp.int32),
        ]
        + [pltpu.VMEM((CH, HW), jnp.float32)] * NB
        + [pltpu.VMEM((NB, CH, HW), jnp.float32)]
        + [pltpu.SemaphoreType.DMA] * (2 * NB),
    )
    def emb(idx_hbm, lut_hbm, out_hbm, idx_v, *rest):
        gbufs, wbuf, sems = rest[:NB], rest[NB], rest[NB + 1:]
        sg, so = sems[:NB], sems[NB:]
        wid = lax.axis_index("s") * 2 + lax.axis_index("c")
        pltpu.sync_copy(idx_hbm.at[wid], idx_v)
        base = wid * (NCH * CH)

        def fire_gather(g, b):
            pltpu.async_copy(lut_hbm.at[idx_v.at[g]], gbufs[b], sg[b])

        def wait_gather(b):
            pltpu.make_async_copy(
                lut_hbm.at[idx_v.at[0]], gbufs[b], sg[b]).wait()

        def fire_out(g, b):
            pltpu.async_copy(
                wbuf.at[b], out_hbm.at[pl.ds(base + g * CH, CH)], so[b])

        def wait_out(b):
            pltpu.make_async_copy(
                wbuf.at[b], out_hbm.at[pl.ds(0, CH)], so[b]).wait()

        def scale(b):
            gref = gbufs[b]

            def grp(r0, c2):
                for u in range(4):
                    r = r0 * 4 + u
                    for h in range(HW // 16):
                        sl = pl.ds(h * 16, 16)
                        wbuf[b, r, sl] = gref[r, sl] * _SCALE
                return c2

            lax.fori_loop(0, CH // 4, grp, 0)

        for b in range(NB):
            fire_gather(b, b)

        for b in range(NB):
            wait_gather(b)
            scale(b)
            fire_out(b, b)
            fire_gather(NB + b, b)

        def round_body(i, carry):
            for b in range(NB):
                g = i * NB + b
                wait_gather(b)
                wait_out(b)
                scale(b)
                fire_out(g, b)
                fire_gather(g + NB, b)
            return carry

        lax.fori_loop(1, R - 1, round_body, 0)

        for b in range(NB):
            g = (R - 1) * NB + b
            wait_gather(b)
            wait_out(b)
            scale(b)
            fire_out(g, b)

        for b in range(NB):
            wait_out(b)

    return emb


def kernel(input, lut):
    nb, nh = input.shape
    B = nb * nh
    ids = input.reshape(-1).astype(jnp.int32)
    hids = ids[:, None] * 2 + jnp.arange(2, dtype=jnp.int32)[None, :]
    hids = hids.reshape(NW, 2 * B // (NW * CH), CH)
    lut2 = lut.reshape(2 * lut.shape[0], HW)
    out = _make_kernel(B)(hids, lut2)
    return out.reshape(nb, nh, D)
